# baseline (speedup 1.0000x reference)
import functools

import jax
import jax.numpy as jnp
from jax import lax
from jax.experimental import pallas as pl
from jax.experimental.pallas import tpu as pltpu
from jax.experimental.pallas import tpu_sc as plsc


def _sc_add(x_hbm, pe_hbm, out_hbm, pe0, pe1, x0, x1, o0, o1,
            x_sem, pe_sem0, pe_sem1, out_sem0, out_sem1, *,
            seq_per_w, n_chunks, chd, batch, seq_d):
    wid = lax.axis_index("s") * 2 + lax.axis_index("c")
    base = wid * seq_per_w * 1024
    total = n_chunks * batch

    pe_bufs = (pe0, pe1)
    x_bufs = (x0, x1)
    o_bufs = (o0, o1)
    pe_sems = (pe_sem0, pe_sem1)
    out_sems = (out_sem0, out_sem1)

    def pe_slice(chunk):
        return pe_hbm.at[pl.ds(base + chunk * chd, chd)]

    def x_slice(t):
        return x_hbm.at[pl.ds((t % batch) * seq_d + base + (t // batch) * chd, chd)]

    def out_slice(t):
        return out_hbm.at[pl.ds((t % batch) * seq_d + base + (t // batch) * chd, chd)]

    pltpu.async_copy(pe_slice(0), pe0, pe_sem0)
    pltpu.async_copy(x_slice(0), x0, x_sem)

    def cp_body(cp, _):
        for cc in range(2):
            chunk = cp * 2 + cc
            peb = pe_bufs[cc]
            for b in range(batch):
                t = chunk * batch + b
                tp = (cc * batch + b) % 2
                xb = x_bufs[tp]
                ob = o_bufs[tp]

                if b == 0:
                    pltpu.make_async_copy(pe_slice(0), peb, pe_sems[cc]).wait()

                    @pl.when(chunk + 1 < n_chunks)
                    def _():
                        pltpu.async_copy(pe_slice(chunk + 1), pe_bufs[1 - cc],
                                         pe_sems[1 - cc])

                pltpu.make_async_copy(x_slice(0), xb, x_sem).wait()

                @pl.when(t + 1 < total)
                def _():
                    pltpu.async_copy(x_slice(t + 1), x_bufs[1 - tp], x_sem)

                @pl.when(t >= 2)
                def _():
                    pltpu.make_async_copy(ob, out_slice(0), out_sems[tp]).wait()

                def add_body(j, _):
                    i = j * 128
                    for k in range(8):
                        off = i + k * 16
                        ob[pl.ds(off, 16)] = (xb[pl.ds(off, 16)]
                                              + peb[pl.ds(off, 16)])
                    return 0

                lax.fori_loop(0, chd // 128, add_body, 0)

                pltpu.async_copy(ob, out_slice(t), out_sems[tp])
        return 0

    lax.fori_loop(0, n_chunks // 2, cp_body, 0)

    pltpu.make_async_copy(o0, out_slice(0), out_sem0).wait()
    pltpu.make_async_copy(o1, out_slice(1), out_sem1).wait()


def _sc_run(x_flat, pe_flat, n_batch, S, D):
    NW = 32
    SEQ_PER_W = S // NW
    CH = 16
    CHD = CH * D
    NCH = SEQ_PER_W // CH

    body = functools.partial(
        _sc_add, seq_per_w=SEQ_PER_W, n_chunks=NCH,
        chd=CHD, batch=n_batch, seq_d=S * D)

    run = pl.kernel(
        body,
        out_type=jax.ShapeDtypeStruct((n_batch * S * D,), jnp.float32),
        mesh=plsc.VectorSubcoreMesh(core_axis_name="c", subcore_axis_name="s"),
        scratch_types=[
            pltpu.VMEM((CHD,), jnp.float32),
            pltpu.VMEM((CHD,), jnp.float32),
            pltpu.VMEM((CHD,), jnp.float32),
            pltpu.VMEM((CHD,), jnp.float32),
            pltpu.VMEM((CHD,), jnp.float32),
            pltpu.VMEM((CHD,), jnp.float32),
            pltpu.SemaphoreType.DMA,
            pltpu.SemaphoreType.DMA,
            pltpu.SemaphoreType.DMA,
            pltpu.SemaphoreType.DMA,
            pltpu.SemaphoreType.DMA,
        ],
    )
    return run(x_flat, pe_flat)


def _tc_body(x_ref, pe_ref, o_ref):
    o_ref[...] = x_ref[...] + pe_ref[...]


def _tc_run(x, pe, b0, n_batch, S, D):
    BS = 512
    grid = (S // BS, n_batch)
    return pl.pallas_call(
        _tc_body,
        grid=grid,
        in_specs=[
            pl.BlockSpec((1, BS, D), lambda s, b: (b + b0, s, 0)),
            pl.BlockSpec((BS, D), lambda s, b: (s, 0)),
        ],
        out_specs=pl.BlockSpec((1, BS, D), lambda s, b: (b, s, 0)),
        out_shape=jax.ShapeDtypeStruct((n_batch, S, D), x.dtype),
    )(x, pe)


def kernel(x, pe_table):
    B, S, D = x.shape
    SC_B = 1
    pe = pe_table[:S]
    sc_out = _sc_run(x.reshape(-1), pe.reshape(-1), SC_B, S, D)
    tc_out = _tc_run(x, pe, SC_B, B - SC_B, S, D)
    return jnp.concatenate([sc_out.reshape(SC_B, S, D), tc_out], axis=0)


# speedup vs baseline: 1.7601x; 1.7601x over previous
import functools

import jax
import jax.numpy as jnp
from jax import lax
from jax.experimental import pallas as pl
from jax.experimental.pallas import tpu as pltpu
from jax.experimental.pallas import tpu_sc as plsc


def _sc_add(x_hbm, pe_hbm, out_hbm, pe0, pe1, x0, x1, o0, o1,
            x_sem, pe_sem0, pe_sem1, out_sem0, out_sem1, *,
            seq_per_w, n_chunks, ch, d, batch, s):
    wid = lax.axis_index("s") * 2 + lax.axis_index("c")
    base = wid * seq_per_w
    total = n_chunks * batch

    pe_bufs = (pe0, pe1)
    x_bufs = (x0, x1)
    o_bufs = (o0, o1)
    pe_sems = (pe_sem0, pe_sem1)
    out_sems = (out_sem0, out_sem1)

    def pe_slice(chunk):
        return pe_hbm.at[pl.ds(base + chunk * ch, ch)]

    def x_slice(t):
        return x_hbm.at[pl.ds((t % batch) * s + base + (t // batch) * ch, ch)]

    def out_slice(t):
        return out_hbm.at[pl.ds((t % batch) * s + base + (t // batch) * ch, ch)]

    pltpu.async_copy(pe_slice(0), pe0, pe_sem0)
    pltpu.async_copy(x_slice(0), x0, x_sem)

    def cp_body(cp, _):
        for cc in range(2):
            chunk = cp * 2 + cc
            peb = pe_bufs[cc]
            for b in range(batch):
                t = chunk * batch + b
                tp = (cc * batch + b) % 2
                xb = x_bufs[tp]
                ob = o_bufs[tp]

                if b == 0:
                    pltpu.make_async_copy(pe_slice(0), peb, pe_sems[cc]).wait()

                    @pl.when(chunk + 1 < n_chunks)
                    def _():
                        pltpu.async_copy(pe_slice(chunk + 1), pe_bufs[1 - cc],
                                         pe_sems[1 - cc])

                pltpu.make_async_copy(x_slice(0), xb, x_sem).wait()

                @pl.when(t + 1 < total)
                def _():
                    pltpu.async_copy(x_slice(t + 1), x_bufs[1 - tp], x_sem)

                @pl.when(t >= 2)
                def _():
                    pltpu.make_async_copy(ob, out_slice(0), out_sems[tp]).wait()

                def add_body(r, _):
                    for c in range(0, d, 16):
                        ob[r, pl.ds(c, 16)] = (xb[r, pl.ds(c, 16)]
                                               + peb[r, pl.ds(c, 16)])
                    return 0

                lax.fori_loop(0, ch, add_body, 0)

                pltpu.async_copy(ob, out_slice(t), out_sems[tp])
        return 0

    lax.fori_loop(0, n_chunks // 2, cp_body, 0)

    pltpu.make_async_copy(o0, out_slice(0), out_sem0).wait()
    pltpu.make_async_copy(o1, out_slice(1), out_sem1).wait()


def _sc_run(x2d, pe, n_batch, S, D):
    NW = 32
    SEQ_PER_W = S // NW
    CH = 16
    NCH = SEQ_PER_W // CH

    body = functools.partial(
        _sc_add, seq_per_w=SEQ_PER_W, n_chunks=NCH,
        ch=CH, d=D, batch=n_batch, s=S)

    run = pl.kernel(
        body,
        out_type=jax.ShapeDtypeStruct((n_batch * S, D), jnp.float32),
        mesh=plsc.VectorSubcoreMesh(core_axis_name="c", subcore_axis_name="s"),
        scratch_types=[
            pltpu.VMEM((CH, D), jnp.float32),
            pltpu.VMEM((CH, D), jnp.float32),
            pltpu.VMEM((CH, D), jnp.float32),
            pltpu.VMEM((CH, D), jnp.float32),
            pltpu.VMEM((CH, D), jnp.float32),
            pltpu.VMEM((CH, D), jnp.float32),
            pltpu.SemaphoreType.DMA,
            pltpu.SemaphoreType.DMA,
            pltpu.SemaphoreType.DMA,
            pltpu.SemaphoreType.DMA,
            pltpu.SemaphoreType.DMA,
        ],
    )
    return run(x2d, pe)


def _tc_body(x_ref, pe_ref, o_ref):
    o_ref[...] = x_ref[...] + pe_ref[...]


def _tc_run(x, pe, b0, n_batch, out_batch, S, D):
    BS = 512
    grid = (S // BS, n_batch)
    return pl.pallas_call(
        _tc_body,
        grid=grid,
        in_specs=[
            pl.BlockSpec((1, BS, D), lambda s, b: (b + b0, s, 0)),
            pl.BlockSpec((BS, D), lambda s, b: (s, 0)),
        ],
        out_specs=pl.BlockSpec((1, BS, D), lambda s, b: (b, s, 0)),
        out_shape=jax.ShapeDtypeStruct((out_batch, S, D), x.dtype),
    )(x, pe)


def kernel(x, pe_table):
    B, S, D = x.shape
    SC_B = 1
    pe = pe_table[:S]
    sc_out = _sc_run(x.reshape(B * S, D), pe, SC_B, S, D)
    tc_out = _tc_run(x, pe, SC_B, B - SC_B, B - SC_B, S, D)
    return jnp.concatenate([sc_out.reshape(SC_B, S, D), tc_out], axis=0)


# speedup vs baseline: 2.4576x; 1.3963x over previous
import functools

import jax
import jax.numpy as jnp
from jax import lax
from jax.experimental import pallas as pl
from jax.experimental.pallas import tpu as pltpu
from jax.experimental.pallas import tpu_sc as plsc


def _sc_add(x_hbm, pe_hbm, out_hbm, pe0, pe1, x0, x1, o0, o1,
            x_sem, pe_sem0, pe_sem1, out_sem0, out_sem1, *,
            seq_per_w, n_chunks, ch, d, batch, s):
    wid = lax.axis_index("s") * 2 + lax.axis_index("c")
    base = wid * seq_per_w
    total = n_chunks * batch

    pe_bufs = (pe0, pe1)
    x_bufs = (x0, x1)
    o_bufs = (o0, o1)
    pe_sems = (pe_sem0, pe_sem1)
    out_sems = (out_sem0, out_sem1)

    def pe_slice(chunk):
        return pe_hbm.at[pl.ds(base + chunk * ch, ch)]

    def x_slice(t):
        return x_hbm.at[pl.ds((t % batch) * s + base + (t // batch) * ch, ch)]

    def out_slice(t):
        return out_hbm.at[pl.ds((t % batch) * s + base + (t // batch) * ch, ch)]

    pltpu.async_copy(pe_slice(0), pe0, pe_sem0)
    pltpu.async_copy(x_slice(0), x0, x_sem)

    def cp_body(cp, _):
        for cc in range(2):
            chunk = cp * 2 + cc
            peb = pe_bufs[cc]
            for b in range(batch):
                t = chunk * batch + b
                tp = (cc * batch + b) % 2
                xb = x_bufs[tp]
                ob = o_bufs[tp]

                if b == 0:
                    pltpu.make_async_copy(pe_slice(0), peb, pe_sems[cc]).wait()

                    @pl.when(chunk + 1 < n_chunks)
                    def _():
                        pltpu.async_copy(pe_slice(chunk + 1), pe_bufs[1 - cc],
                                         pe_sems[1 - cc])

                pltpu.make_async_copy(x_slice(0), xb, x_sem).wait()

                @pl.when(t + 1 < total)
                def _():
                    pltpu.async_copy(x_slice(t + 1), x_bufs[1 - tp], x_sem)

                @pl.when(t >= 2)
                def _():
                    pltpu.make_async_copy(ob, out_slice(0), out_sems[tp]).wait()

                def add_body(r, _):
                    for c in range(0, d, 16):
                        ob[r, pl.ds(c, 16)] = (xb[r, pl.ds(c, 16)]
                                               + peb[r, pl.ds(c, 16)])
                    return 0

                lax.fori_loop(0, ch, add_body, 0)

                pltpu.async_copy(ob, out_slice(t), out_sems[tp])
        return 0

    lax.fori_loop(0, n_chunks // 2, cp_body, 0)

    pltpu.make_async_copy(o0, out_slice(0), out_sem0).wait()
    pltpu.make_async_copy(o1, out_slice(1), out_sem1).wait()


def _sc_run(x2d, pe, n_batch, S, D):
    NW = 32
    SEQ_PER_W = S // NW
    CH = 16
    NCH = SEQ_PER_W // CH

    body = functools.partial(
        _sc_add, seq_per_w=SEQ_PER_W, n_chunks=NCH,
        ch=CH, d=D, batch=n_batch, s=S)

    run = pl.kernel(
        body,
        out_type=jax.ShapeDtypeStruct((n_batch * S, D), jnp.float32),
        mesh=plsc.VectorSubcoreMesh(core_axis_name="c", subcore_axis_name="s"),
        scratch_types=[
            pltpu.VMEM((CH, D), jnp.float32),
            pltpu.VMEM((CH, D), jnp.float32),
            pltpu.VMEM((CH, D), jnp.float32),
            pltpu.VMEM((CH, D), jnp.float32),
            pltpu.VMEM((CH, D), jnp.float32),
            pltpu.VMEM((CH, D), jnp.float32),
            pltpu.SemaphoreType.DMA,
            pltpu.SemaphoreType.DMA,
            pltpu.SemaphoreType.DMA,
            pltpu.SemaphoreType.DMA,
            pltpu.SemaphoreType.DMA,
        ],
    )
    return run(x2d, pe)


def _tc_body(x_ref, pe_ref, o_ref):
    o_ref[...] = x_ref[...] + pe_ref[...]


def _tc_run(x, pe, b0, n_batch, out_batch, S, D):
    BS = 512
    grid = (S // BS, n_batch)
    return pl.pallas_call(
        _tc_body,
        grid=grid,
        in_specs=[
            pl.BlockSpec((1, BS, D), lambda s, b: (b + b0, s, 0)),
            pl.BlockSpec((BS, D), lambda s, b: (s, 0)),
        ],
        out_specs=pl.BlockSpec((1, BS, D), lambda s, b: (b + b0, s, 0)),
        out_shape=jax.ShapeDtypeStruct((out_batch, S, D), x.dtype),
    )(x, pe)


def kernel(x, pe_table):
    B, S, D = x.shape
    SC_B = 1
    pe = pe_table[:S]
    sc_out = _sc_run(x.reshape(B * S, D), pe, SC_B, S, D)
    tc_out = _tc_run(x, pe, SC_B, B - SC_B, B, S, D)
    sc_out, tc_out = lax.optimization_barrier((sc_out, tc_out))
    return lax.dynamic_update_slice(
        tc_out, sc_out.reshape(SC_B, S, D), (0, 0, 0))


# speedup vs baseline: 2.5220x; 1.0262x over previous
import functools

import jax
import jax.numpy as jnp
from jax import lax
from jax.experimental import pallas as pl
from jax.experimental.pallas import tpu as pltpu
from jax.experimental.pallas import tpu_sc as plsc


def _sc_add(x_hbm, pe_hbm, out_hbm, pe0, pe1, x0, x1, o0, o1,
            x_sem, pe_sem0, pe_sem1, out_sem0, out_sem1, *,
            seq_per_w, n_chunks, ch, d, batch, s):
    wid = lax.axis_index("s") * 2 + lax.axis_index("c")
    base = wid * seq_per_w
    total = n_chunks * batch

    pe_bufs = (pe0, pe1)
    x_bufs = (x0, x1)
    o_bufs = (o0, o1)
    pe_sems = (pe_sem0, pe_sem1)
    out_sems = (out_sem0, out_sem1)

    def pe_slice(chunk):
        return pe_hbm.at[pl.ds(base + chunk * ch, ch)]

    def x_slice(t):
        return x_hbm.at[pl.ds((t % batch) * s + base + (t // batch) * ch, ch)]

    def out_slice(t):
        return out_hbm.at[pl.ds((t % batch) * s + base + (t // batch) * ch, ch)]

    pltpu.async_copy(pe_slice(0), pe0, pe_sem0)
    pltpu.async_copy(x_slice(0), x0, x_sem)

    def cp_body(cp, _):
        for cc in range(2):
            chunk = cp * 2 + cc
            peb = pe_bufs[cc]
            for b in range(batch):
                t = chunk * batch + b
                tp = (cc * batch + b) % 2
                xb = x_bufs[tp]
                ob = o_bufs[tp]

                if b == 0:
                    pltpu.make_async_copy(pe_slice(0), peb, pe_sems[cc]).wait()

                    @pl.when(chunk + 1 < n_chunks)
                    def _():
                        pltpu.async_copy(pe_slice(chunk + 1), pe_bufs[1 - cc],
                                         pe_sems[1 - cc])

                pltpu.make_async_copy(x_slice(0), xb, x_sem).wait()

                @pl.when(t + 1 < total)
                def _():
                    pltpu.async_copy(x_slice(t + 1), x_bufs[1 - tp], x_sem)

                @pl.when(t >= 2)
                def _():
                    pltpu.make_async_copy(ob, out_slice(0), out_sems[tp]).wait()

                def add_body(r, _):
                    for c in range(0, d, 16):
                        ob[r, pl.ds(c, 16)] = (xb[r, pl.ds(c, 16)]
                                               + peb[r, pl.ds(c, 16)])
                    return 0

                lax.fori_loop(0, ch, add_body, 0)

                pltpu.async_copy(ob, out_slice(t), out_sems[tp])
        return 0

    lax.fori_loop(0, n_chunks // 2, cp_body, 0)

    pltpu.make_async_copy(o0, out_slice(0), out_sem0).wait()
    pltpu.make_async_copy(o1, out_slice(1), out_sem1).wait()


def _sc_run(x2d, pe, n_batch, S, D):
    NW = 32
    SEQ_PER_W = S // NW
    CH = 16
    NCH = SEQ_PER_W // CH

    body = functools.partial(
        _sc_add, seq_per_w=SEQ_PER_W, n_chunks=NCH,
        ch=CH, d=D, batch=n_batch, s=S)

    run = pl.kernel(
        body,
        out_type=jax.ShapeDtypeStruct((n_batch * S, D), jnp.float32),
        mesh=plsc.VectorSubcoreMesh(core_axis_name="c", subcore_axis_name="s"),
        scratch_types=[
            pltpu.VMEM((CH, D), jnp.float32),
            pltpu.VMEM((CH, D), jnp.float32),
            pltpu.VMEM((CH, D), jnp.float32),
            pltpu.VMEM((CH, D), jnp.float32),
            pltpu.VMEM((CH, D), jnp.float32),
            pltpu.VMEM((CH, D), jnp.float32),
            pltpu.SemaphoreType.DMA,
            pltpu.SemaphoreType.DMA,
            pltpu.SemaphoreType.DMA,
            pltpu.SemaphoreType.DMA,
            pltpu.SemaphoreType.DMA,
        ],
    )
    return run(x2d, pe)


def _tc_body(x_ref, pe_ref, o_ref):
    o_ref[...] = x_ref[...] + pe_ref[...]


def _tc_run(x, pe, b0, n_batch, out_batch, S, D):
    BS = 1024
    grid = (S // BS, n_batch)
    return pl.pallas_call(
        _tc_body,
        grid=grid,
        in_specs=[
            pl.BlockSpec((1, BS, D), lambda s, b: (b + b0, s, 0)),
            pl.BlockSpec((BS, D), lambda s, b: (s, 0)),
        ],
        out_specs=pl.BlockSpec((1, BS, D), lambda s, b: (b + b0, s, 0)),
        out_shape=jax.ShapeDtypeStruct((out_batch, S, D), x.dtype),
    )(x, pe)


def kernel(x, pe_table):
    B, S, D = x.shape
    SC_B = 1
    pe = pe_table[:S]
    sc_out = _sc_run(x.reshape(B * S, D), pe, SC_B, S, D)
    tc_out = _tc_run(x, pe, SC_B, B - SC_B, B, S, D)
    sc_out, tc_out = lax.optimization_barrier((sc_out, tc_out))
    return lax.dynamic_update_slice(
        tc_out, sc_out.reshape(SC_B, S, D), (0, 0, 0))


# speedup vs baseline: 2.5757x; 1.0213x over previous
import functools

import jax
import jax.numpy as jnp
from jax import lax
from jax.experimental import pallas as pl
from jax.experimental.pallas import tpu as pltpu
from jax.experimental.pallas import tpu_sc as plsc


def _sc_add(x_hbm, pe_hbm, out_hbm, pe0, pe1, x0, x1, o0, o1,
            x_sem, pe_sem0, pe_sem1, out_sem0, out_sem1, *,
            seq_per_w, n_chunks, ch, d, batch, s):
    wid = lax.axis_index("s") * 2 + lax.axis_index("c")
    base = wid * seq_per_w
    total = n_chunks * batch

    pe_bufs = (pe0, pe1)
    x_bufs = (x0, x1)
    o_bufs = (o0, o1)
    pe_sems = (pe_sem0, pe_sem1)
    out_sems = (out_sem0, out_sem1)

    def pe_slice(chunk):
        return pe_hbm.at[pl.ds(base + chunk * ch, ch)]

    def x_slice(t):
        return x_hbm.at[pl.ds((t % batch) * s + base + (t // batch) * ch, ch)]

    def out_slice(t):
        return out_hbm.at[pl.ds((t % batch) * s + base + (t // batch) * ch, ch)]

    pltpu.async_copy(pe_slice(0), pe0, pe_sem0)
    pltpu.async_copy(x_slice(0), x0, x_sem)

    def cp_body(cp, _):
        for cc in range(2):
            chunk = cp * 2 + cc
            peb = pe_bufs[cc]
            for b in range(batch):
                t = chunk * batch + b
                tp = (cc * batch + b) % 2
                xb = x_bufs[tp]
                ob = o_bufs[tp]

                if b == 0:
                    pltpu.make_async_copy(pe_slice(0), peb, pe_sems[cc]).wait()

                    @pl.when(chunk + 1 < n_chunks)
                    def _():
                        pltpu.async_copy(pe_slice(chunk + 1), pe_bufs[1 - cc],
                                         pe_sems[1 - cc])

                pltpu.make_async_copy(x_slice(0), xb, x_sem).wait()

                @pl.when(t + 1 < total)
                def _():
                    pltpu.async_copy(x_slice(t + 1), x_bufs[1 - tp], x_sem)

                @pl.when(t >= 2)
                def _():
                    pltpu.make_async_copy(ob, out_slice(0), out_sems[tp]).wait()

                def add_body(r, _):
                    for c in range(0, d, 16):
                        ob[r, pl.ds(c, 16)] = (xb[r, pl.ds(c, 16)]
                                               + peb[r, pl.ds(c, 16)])
                    return 0

                lax.fori_loop(0, ch, add_body, 0)

                pltpu.async_copy(ob, out_slice(t), out_sems[tp])
        return 0

    lax.fori_loop(0, n_chunks // 2, cp_body, 0)

    pltpu.make_async_copy(o0, out_slice(0), out_sem0).wait()
    pltpu.make_async_copy(o1, out_slice(1), out_sem1).wait()


def _sc_run(x2d, pe, n_batch, S, D):
    NW = 32
    SEQ_PER_W = S // NW
    CH = 16
    NCH = SEQ_PER_W // CH

    body = functools.partial(
        _sc_add, seq_per_w=SEQ_PER_W, n_chunks=NCH,
        ch=CH, d=D, batch=n_batch, s=S)

    run = pl.kernel(
        body,
        out_type=jax.ShapeDtypeStruct((n_batch * S, D), jnp.float32),
        mesh=plsc.VectorSubcoreMesh(core_axis_name="c", subcore_axis_name="s"),
        scratch_types=[
            pltpu.VMEM((CH, D), jnp.float32),
            pltpu.VMEM((CH, D), jnp.float32),
            pltpu.VMEM((CH, D), jnp.float32),
            pltpu.VMEM((CH, D), jnp.float32),
            pltpu.VMEM((CH, D), jnp.float32),
            pltpu.VMEM((CH, D), jnp.float32),
            pltpu.SemaphoreType.DMA,
            pltpu.SemaphoreType.DMA,
            pltpu.SemaphoreType.DMA,
            pltpu.SemaphoreType.DMA,
            pltpu.SemaphoreType.DMA,
        ],
    )
    return run(x2d, pe)


def _tc_body(x_ref, pe_ref, o_ref):
    o_ref[...] = x_ref[...] + pe_ref[...]


def _tc_run(x, pe, b0, n_batch, out_batch, S, D):
    BS = 2048
    grid = (S // BS, n_batch)
    return pl.pallas_call(
        _tc_body,
        grid=grid,
        in_specs=[
            pl.BlockSpec((1, BS, D), lambda s, b: (b + b0, s, 0)),
            pl.BlockSpec((BS, D), lambda s, b: (s, 0)),
        ],
        out_specs=pl.BlockSpec((1, BS, D), lambda s, b: (b + b0, s, 0)),
        out_shape=jax.ShapeDtypeStruct((out_batch, S, D), x.dtype),
    )(x, pe)


def kernel(x, pe_table):
    B, S, D = x.shape
    SC_B = 1
    pe = pe_table[:S]
    sc_out = _sc_run(x.reshape(B * S, D), pe, SC_B, S, D)
    tc_out = _tc_run(x, pe, SC_B, B - SC_B, B, S, D)
    sc_out, tc_out = lax.optimization_barrier((sc_out, tc_out))
    return lax.dynamic_update_slice(
        tc_out, sc_out.reshape(SC_B, S, D), (0, 0, 0))
